# Initial kernel scaffold; baseline (speedup 1.0000x reference)
import jax
import jax.numpy as jnp
from jax.experimental import pallas as pl


def kernel(image, dvf):
    raise NotImplementedError("write your pallas kernel here")


# speedup vs baseline: 1.0449x; 1.0449x over previous
import jax
import jax.numpy as jnp
from jax import lax
from jax.experimental import pallas as pl
from jax.experimental.pallas import tpu as pltpu
from jax.experimental.pallas import tpu_sc as plsc

H = W = D = 128
VOX = H * W * D
NC, NS, L = 2, 16, 16
NW = NC * NS
PER_W = VOX // NW
SUP = 1024
NSUP = PER_W // SUP
LINES = SUP // D
VPL = D // L

_SHIFTS = (0, 1, D, D + 1, W * D, W * D + 1, W * D + D, W * D + D + 1)


def _body(img_ref, dy_ref, dx_ref, dz_ref, out_ref,
          dyv, dxv, dzv, idx_v, ywv, xwv, zwv, gat, out_v,
          dsem, gsem, osem):
    cid = lax.axis_index("c")
    sid = lax.axis_index("s")
    wid = sid * NC + cid
    base = wid * PER_W

    def floor_clip(nf, lim):
        t = nf.astype(jnp.int32)
        fl = jnp.where(nf < t.astype(jnp.float32), t - 1, t)
        c = jnp.minimum(jnp.maximum(fl, 0), lim - 2)
        w = jnp.minimum(jnp.maximum(nf - c.astype(jnp.float32), 0.0), 1.0)
        return c, w

    @pl.loop(0, NSUP)
    def _sup(s):
        sbase = base + s * SUP

        cpy = pltpu.async_copy(dy_ref.at[pl.ds(sbase, SUP)], dyv, dsem)
        cpx = pltpu.async_copy(dx_ref.at[pl.ds(sbase, SUP)], dxv, dsem)
        cpz = pltpu.async_copy(dz_ref.at[pl.ds(sbase, SUP)], dzv, dsem)
        cpy.wait()
        cpx.wait()
        cpz.wait()

        gl0 = wid * (PER_W // D) + s * LINES

        @pl.loop(0, LINES)
        def _idx(li):
            gl = gl0 + li
            lane = lax.iota(jnp.int32, L)
            yy = (gl >> 7).astype(jnp.float32)
            xx = (gl & 127).astype(jnp.float32)
            for v in range(VPL):
                o = li * D + v * L
                zz = (lane + v * L).astype(jnp.float32)
                ny = dyv[pl.ds(o, L)] + yy
                nx = dxv[pl.ds(o, L)] + xx
                nz = dzv[pl.ds(o, L)] + zz
                yc, yw = floor_clip(ny, H)
                xc, xw = floor_clip(nx, W)
                zc, zw = floor_clip(nz, D)
                f000 = yc * (W * D) + xc * D + zc
                for k, sh in enumerate(_SHIFTS):
                    idx_v[pl.ds(k * SUP + o, L)] = f000 + sh
                ywv[pl.ds(o, L)] = yw
                xwv[pl.ds(o, L)] = xw
                zwv[pl.ds(o, L)] = zw

        @pl.loop(0, LINES)
        def _fire(li):
            for k in range(8):
                pltpu.async_copy(
                    img_ref.at[idx_v.at[pl.ds(k * SUP + li * D, D)]],
                    gat.at[pl.ds(k * SUP + li * D, D)], gsem)

        for k in range(8):
            pltpu.make_async_copy(dy_ref.at[pl.ds(0, SUP)],
                                  gat.at[pl.ds(k * SUP, SUP)], gsem).wait()

        @pl.loop(0, LINES)
        def _combine(li):
            for v in range(VPL):
                o = li * D + v * L
                cv = [gat[pl.ds(k * SUP + o, L)] for k in range(8)]
                yw = ywv[pl.ds(o, L)]
                xw = xwv[pl.ds(o, L)]
                zw = zwv[pl.ds(o, L)]
                c00 = cv[0] + zw * (cv[1] - cv[0])
                c01 = cv[2] + zw * (cv[3] - cv[2])
                c10 = cv[4] + zw * (cv[5] - cv[4])
                c11 = cv[6] + zw * (cv[7] - cv[6])
                c0 = c00 + xw * (c01 - c00)
                c1 = c10 + xw * (c11 - c10)
                out_v[pl.ds(o, L)] = c0 + yw * (c1 - c0)

        pltpu.async_copy(out_v, out_ref.at[pl.ds(sbase, SUP)], osem).wait()


_warp = pl.kernel(
    _body,
    out_type=jax.ShapeDtypeStruct((VOX,), jnp.float32),
    mesh=plsc.VectorSubcoreMesh(core_axis_name="c", subcore_axis_name="s"),
    scratch_types=[
        pltpu.VMEM((SUP,), jnp.float32),
        pltpu.VMEM((SUP,), jnp.float32),
        pltpu.VMEM((SUP,), jnp.float32),
        pltpu.VMEM((8 * SUP,), jnp.int32),
        pltpu.VMEM((SUP,), jnp.float32),
        pltpu.VMEM((SUP,), jnp.float32),
        pltpu.VMEM((SUP,), jnp.float32),
        pltpu.VMEM((8 * SUP,), jnp.float32),
        pltpu.VMEM((SUP,), jnp.float32),
        pltpu.SemaphoreType.DMA,
        pltpu.SemaphoreType.DMA,
        pltpu.SemaphoreType.DMA,
    ],
)


def kernel(image, dvf):
    flat = image.reshape(VOX)
    dy = dvf[0, 0].reshape(VOX)
    dx = dvf[0, 1].reshape(VOX)
    dz = dvf[0, 2].reshape(VOX)
    out = _warp(flat, dy, dx, dz)
    return out.reshape(1, 1, 1, H, W, D)


# speedup vs baseline: 1.1490x; 1.0996x over previous
import jax
import jax.numpy as jnp
from jax import lax
from jax.experimental import pallas as pl
from jax.experimental.pallas import tpu as pltpu
from jax.experimental.pallas import tpu_sc as plsc

H = W = D = 128
VOX = H * W * D
NC, NS, L = 2, 16, 16
NW = NC * NS
PER_W = VOX // NW
SUP = 1024
NSUP = PER_W // SUP
LINES = SUP // D
VPL = D // L

_SHIFTS = (0, 1, D, D + 1, W * D, W * D + 1, W * D + D, W * D + D + 1)


def _body(img_ref, dy_ref, dx_ref, dz_ref, out_ref,
          dyv, dxv, dzv, idx_v, ywv, xwv, zwv, gat, out_v,
          dsem, gsem, osem):
    cid = lax.axis_index("c")
    sid = lax.axis_index("s")
    wid = sid * NC + cid
    base = wid * PER_W

    def floor_clip(nf, lim):
        t = nf.astype(jnp.int32)
        fl = jnp.where(nf < t.astype(jnp.float32), t - 1, t)
        c = jnp.minimum(jnp.maximum(fl, 0), lim - 2)
        w = jnp.minimum(jnp.maximum(nf - c.astype(jnp.float32), 0.0), 1.0)
        return c, w

    def fire_dvf(s, b):
        sbase = base + s * SUP
        pltpu.async_copy(dy_ref.at[pl.ds(sbase, SUP)], dyv[b], dsem)
        pltpu.async_copy(dx_ref.at[pl.ds(sbase, SUP)], dxv[b], dsem)
        pltpu.async_copy(dz_ref.at[pl.ds(sbase, SUP)], dzv[b], dsem)

    def wait_dvf(b):
        pltpu.make_async_copy(dy_ref.at[pl.ds(0, SUP)], dyv[b], dsem).wait()
        pltpu.make_async_copy(dy_ref.at[pl.ds(0, SUP)], dxv[b], dsem).wait()
        pltpu.make_async_copy(dy_ref.at[pl.ds(0, SUP)], dzv[b], dsem).wait()

    def compute_idx(s, b):
        gl0 = wid * (PER_W // D) + s * LINES

        @pl.loop(0, LINES)
        def _idx(li):
            gl = gl0 + li
            lane = lax.iota(jnp.int32, L)
            yy = (gl >> 7).astype(jnp.float32)
            xx = (gl & 127).astype(jnp.float32)
            for v in range(VPL):
                o = li * D + v * L
                zz = (lane + v * L).astype(jnp.float32)
                ny = dyv[b][pl.ds(o, L)] + yy
                nx = dxv[b][pl.ds(o, L)] + xx
                nz = dzv[b][pl.ds(o, L)] + zz
                yc, yw = floor_clip(ny, H)
                xc, xw = floor_clip(nx, W)
                zc, zw = floor_clip(nz, D)
                f000 = yc * (W * D) + xc * D + zc
                for k, sh in enumerate(_SHIFTS):
                    idx_v[b][pl.ds(k * SUP + o, L)] = f000 + sh
                ywv[b][pl.ds(o, L)] = yw
                xwv[b][pl.ds(o, L)] = xw
                zwv[b][pl.ds(o, L)] = zw

    def fire_gathers(b):
        @pl.loop(0, LINES)
        def _fire(li):
            for k in range(8):
                pltpu.async_copy(
                    img_ref.at[idx_v[b].at[pl.ds(k * SUP + li * D, D)]],
                    gat[b].at[pl.ds(k * SUP + li * D, D)], gsem[b])

    def wait_gathers(b):
        for k in range(8):
            pltpu.make_async_copy(dy_ref.at[pl.ds(0, SUP)],
                                  gat[b].at[pl.ds(k * SUP, SUP)], gsem[b]).wait()

    def combine(s, b):
        @pl.loop(0, LINES)
        def _combine(li):
            for v in range(VPL):
                o = li * D + v * L
                cv = [gat[b][pl.ds(k * SUP + o, L)] for k in range(8)]
                yw = ywv[b][pl.ds(o, L)]
                xw = xwv[b][pl.ds(o, L)]
                zw = zwv[b][pl.ds(o, L)]
                c00 = cv[0] + zw * (cv[1] - cv[0])
                c01 = cv[2] + zw * (cv[3] - cv[2])
                c10 = cv[4] + zw * (cv[5] - cv[4])
                c11 = cv[6] + zw * (cv[7] - cv[6])
                c0 = c00 + xw * (c01 - c00)
                c1 = c10 + xw * (c11 - c10)
                out_v[b][pl.ds(o, L)] = c0 + yw * (c1 - c0)

        pltpu.async_copy(out_v[b], out_ref.at[pl.ds(base + s * SUP, SUP)],
                         osem[b])

    def wait_out(b):
        pltpu.make_async_copy(dy_ref.at[pl.ds(0, SUP)], out_v[b],
                              osem[b]).wait()

    fire_dvf(0, 0)

    @pl.loop(0, NSUP // 2 + 1)
    def _t(t):
        for b in range(2):
            s = t * 2 + b

            @pl.when(s < NSUP)
            def _launch():
                wait_dvf(b)
                compute_idx(s, b)
                fire_gathers(b)

            @pl.when(s + 1 < NSUP)
            def _prefetch():
                fire_dvf(s + 1, 1 - b)

            @pl.when(s >= 3)
            def _drain_out():
                wait_out(1 - b)

            @pl.when(jnp.logical_and(s >= 1, s - 1 < NSUP))
            def _finish():
                wait_gathers(1 - b)
                combine(s - 1, 1 - b)

    wait_out((NSUP - 1) & 1)


def _mk_scratch():
    return [
        [pltpu.VMEM((SUP,), jnp.float32)] * 2,
        [pltpu.VMEM((SUP,), jnp.float32)] * 2,
        [pltpu.VMEM((SUP,), jnp.float32)] * 2,
        [pltpu.VMEM((8 * SUP,), jnp.int32)] * 2,
        [pltpu.VMEM((SUP,), jnp.float32)] * 2,
        [pltpu.VMEM((SUP,), jnp.float32)] * 2,
        [pltpu.VMEM((SUP,), jnp.float32)] * 2,
        [pltpu.VMEM((8 * SUP,), jnp.float32)] * 2,
        [pltpu.VMEM((SUP,), jnp.float32)] * 2,
        pltpu.SemaphoreType.DMA,
        [pltpu.SemaphoreType.DMA] * 2,
        [pltpu.SemaphoreType.DMA] * 2,
    ]


_warp = pl.kernel(
    _body,
    out_type=jax.ShapeDtypeStruct((VOX,), jnp.float32),
    mesh=plsc.VectorSubcoreMesh(core_axis_name="c", subcore_axis_name="s"),
    scratch_types=_mk_scratch(),
)


def kernel(image, dvf):
    flat = image.reshape(VOX)
    dy = dvf[0, 0].reshape(VOX)
    dx = dvf[0, 1].reshape(VOX)
    dz = dvf[0, 2].reshape(VOX)
    out = _warp(flat, dy, dx, dz)
    return out.reshape(1, 1, 1, H, W, D)


# speedup vs baseline: 1.3283x; 1.1561x over previous
import jax
import jax.numpy as jnp
from jax import lax
from jax.experimental import pallas as pl
from jax.experimental.pallas import tpu as pltpu
from jax.experimental.pallas import tpu_sc as plsc

H = W = D = 128
VOX = H * W * D
NC, NS, L = 2, 16, 16
NW = NC * NS
PER_W = VOX // NW
SUP = 1024
NSUP = PER_W // SUP
LINES = SUP // D
VPL = D // L

_SHIFTS = (0, 1, D, D + 1, W * D, W * D + 1, W * D + D, W * D + D + 1)


def _body(img_ref, dvf_ref, out_ref,
          dyv, dxv, dzv, idx_v, ywv, xwv, zwv, gat, out_v,
          dsem, gsem, osem):
    cid = lax.axis_index("c")
    sid = lax.axis_index("s")
    wid = sid * NC + cid
    base = wid * PER_W

    def floor_clip(nf, lim):
        t = nf.astype(jnp.int32)
        fl = jnp.where(nf < t.astype(jnp.float32), t - 1, t)
        c = jnp.minimum(jnp.maximum(fl, 0), lim - 2)
        w = jnp.minimum(jnp.maximum(nf - c.astype(jnp.float32), 0.0), 1.0)
        return c, w

    def fire_dvf(s, b):
        sbase = base + s * SUP
        pltpu.async_copy(dvf_ref.at[pl.ds(sbase, SUP)], dyv[b], dsem)
        pltpu.async_copy(dvf_ref.at[pl.ds(VOX + sbase, SUP)], dxv[b], dsem)
        pltpu.async_copy(dvf_ref.at[pl.ds(2 * VOX + sbase, SUP)], dzv[b], dsem)

    def wait_dvf(b):
        pltpu.make_async_copy(dvf_ref.at[pl.ds(0, SUP)], dyv[b], dsem).wait()
        pltpu.make_async_copy(dvf_ref.at[pl.ds(0, SUP)], dxv[b], dsem).wait()
        pltpu.make_async_copy(dvf_ref.at[pl.ds(0, SUP)], dzv[b], dsem).wait()

    def compute_idx(s, b):
        gl0 = wid * (PER_W // D) + s * LINES

        @pl.loop(0, LINES)
        def _idx(li):
            gl = gl0 + li
            lane = lax.iota(jnp.int32, L)
            yy = (gl >> 7).astype(jnp.float32)
            xx = (gl & 127).astype(jnp.float32)
            for v in range(VPL):
                o = li * D + v * L
                zz = (lane + v * L).astype(jnp.float32)
                ny = dyv[b][pl.ds(o, L)] + yy
                nx = dxv[b][pl.ds(o, L)] + xx
                nz = dzv[b][pl.ds(o, L)] + zz
                yc, yw = floor_clip(ny, H)
                xc, xw = floor_clip(nx, W)
                zc, zw = floor_clip(nz, D)
                f000 = yc * (W * D) + xc * D + zc
                for k, sh in enumerate(_SHIFTS):
                    idx_v[b][pl.ds(k * SUP + o, L)] = f000 + sh
                ywv[b][pl.ds(o, L)] = yw
                xwv[b][pl.ds(o, L)] = xw
                zwv[b][pl.ds(o, L)] = zw

    def fire_gathers(b):
        for k in range(8):
            pltpu.async_copy(
                img_ref.at[idx_v[b].at[pl.ds(k * SUP, SUP)]],
                gat[b].at[pl.ds(k * SUP, SUP)], gsem[b])

    def wait_gathers(b):
        for k in range(8):
            pltpu.make_async_copy(dvf_ref.at[pl.ds(0, SUP)],
                                  gat[b].at[pl.ds(k * SUP, SUP)], gsem[b]).wait()

    def combine(s, b):
        @pl.loop(0, LINES)
        def _combine(li):
            for v in range(VPL):
                o = li * D + v * L
                cv = [gat[b][pl.ds(k * SUP + o, L)] for k in range(8)]
                yw = ywv[b][pl.ds(o, L)]
                xw = xwv[b][pl.ds(o, L)]
                zw = zwv[b][pl.ds(o, L)]
                c00 = cv[0] + zw * (cv[1] - cv[0])
                c01 = cv[2] + zw * (cv[3] - cv[2])
                c10 = cv[4] + zw * (cv[5] - cv[4])
                c11 = cv[6] + zw * (cv[7] - cv[6])
                c0 = c00 + xw * (c01 - c00)
                c1 = c10 + xw * (c11 - c10)
                out_v[b][pl.ds(o, L)] = c0 + yw * (c1 - c0)

        pltpu.async_copy(out_v[b], out_ref.at[pl.ds(base + s * SUP, SUP)],
                         osem[b])

    def wait_out(b):
        pltpu.make_async_copy(dvf_ref.at[pl.ds(0, SUP)], out_v[b],
                              osem[b]).wait()

    fire_dvf(0, 0)

    @pl.loop(0, NSUP // 2 + 1)
    def _t(t):
        for b in range(2):
            s = t * 2 + b

            @pl.when(s < NSUP)
            def _launch():
                wait_dvf(b)
                compute_idx(s, b)
                fire_gathers(b)

            @pl.when(s + 1 < NSUP)
            def _prefetch():
                fire_dvf(s + 1, 1 - b)

            @pl.when(s >= 3)
            def _drain_out():
                wait_out(1 - b)

            @pl.when(jnp.logical_and(s >= 1, s - 1 < NSUP))
            def _finish():
                wait_gathers(1 - b)
                combine(s - 1, 1 - b)

    wait_out((NSUP - 1) & 1)


def _mk_scratch():
    return [
        [pltpu.VMEM((SUP,), jnp.float32)] * 2,
        [pltpu.VMEM((SUP,), jnp.float32)] * 2,
        [pltpu.VMEM((SUP,), jnp.float32)] * 2,
        [pltpu.VMEM((8 * SUP,), jnp.int32)] * 2,
        [pltpu.VMEM((SUP,), jnp.float32)] * 2,
        [pltpu.VMEM((SUP,), jnp.float32)] * 2,
        [pltpu.VMEM((SUP,), jnp.float32)] * 2,
        [pltpu.VMEM((8 * SUP,), jnp.float32)] * 2,
        [pltpu.VMEM((SUP,), jnp.float32)] * 2,
        pltpu.SemaphoreType.DMA,
        [pltpu.SemaphoreType.DMA] * 2,
        [pltpu.SemaphoreType.DMA] * 2,
    ]


_warp = pl.kernel(
    _body,
    out_type=jax.ShapeDtypeStruct((VOX,), jnp.float32),
    mesh=plsc.VectorSubcoreMesh(core_axis_name="c", subcore_axis_name="s"),
    scratch_types=_mk_scratch(),
)


def kernel(image, dvf):
    flat = image.reshape(VOX)
    dvf_flat = dvf.reshape(3 * VOX)
    out = _warp(flat, dvf_flat)
    return out.reshape(1, 1, 1, H, W, D)


# speedup vs baseline: 1.4075x; 1.0596x over previous
import jax
import jax.numpy as jnp
from jax import lax
from jax.experimental import pallas as pl
from jax.experimental.pallas import tpu as pltpu
from jax.experimental.pallas import tpu_sc as plsc

H = W = D = 128
VOX = H * W * D
NC, NS, L = 2, 16, 16
NW = NC * NS
PER_W = VOX // NW
SUP = 2048
NSUP = PER_W // SUP
LINES = SUP // D
VPL = D // L

_SHIFTS = (0, 1, D, D + 1, W * D, W * D + 1, W * D + D, W * D + D + 1)


def _body(img_ref, dvf_ref, out_ref,
          dyv, dxv, dzv, idx_v, ywv, xwv, zwv, gat, out_v,
          dsem, gsem, osem):
    cid = lax.axis_index("c")
    sid = lax.axis_index("s")
    wid = sid * NC + cid
    base = wid * PER_W

    def floor_clip(nf, lim):
        t = nf.astype(jnp.int32)
        fl = jnp.where(nf < t.astype(jnp.float32), t - 1, t)
        c = jnp.minimum(jnp.maximum(fl, 0), lim - 2)
        w = jnp.minimum(jnp.maximum(nf - c.astype(jnp.float32), 0.0), 1.0)
        return c, w

    def fire_dvf(s, b):
        sbase = base + s * SUP
        pltpu.async_copy(dvf_ref.at[pl.ds(sbase, SUP)], dyv[b], dsem)
        pltpu.async_copy(dvf_ref.at[pl.ds(VOX + sbase, SUP)], dxv[b], dsem)
        pltpu.async_copy(dvf_ref.at[pl.ds(2 * VOX + sbase, SUP)], dzv[b], dsem)

    def wait_dvf(b):
        pltpu.make_async_copy(dvf_ref.at[pl.ds(0, SUP)], dyv[b], dsem).wait()
        pltpu.make_async_copy(dvf_ref.at[pl.ds(0, SUP)], dxv[b], dsem).wait()
        pltpu.make_async_copy(dvf_ref.at[pl.ds(0, SUP)], dzv[b], dsem).wait()

    def compute_idx(s, b):
        gl0 = wid * (PER_W // D) + s * LINES

        @pl.loop(0, LINES)
        def _idx(li):
            gl = gl0 + li
            lane = lax.iota(jnp.int32, L)
            yy = (gl >> 7).astype(jnp.float32)
            xx = (gl & 127).astype(jnp.float32)
            for v in range(VPL):
                o = li * D + v * L
                zz = (lane + v * L).astype(jnp.float32)
                ny = dyv[b][pl.ds(o, L)] + yy
                nx = dxv[b][pl.ds(o, L)] + xx
                nz = dzv[b][pl.ds(o, L)] + zz
                yc, yw = floor_clip(ny, H)
                xc, xw = floor_clip(nx, W)
                zc, zw = floor_clip(nz, D)
                f000 = yc * (W * D) + xc * D + zc
                for k, sh in enumerate(_SHIFTS):
                    idx_v[b][pl.ds(k * SUP + o, L)] = f000 + sh
                ywv[b][pl.ds(o, L)] = yw
                xwv[b][pl.ds(o, L)] = xw
                zwv[b][pl.ds(o, L)] = zw

    def fire_gathers(b):
        for k in range(8):
            pltpu.async_copy(
                img_ref.at[idx_v[b].at[pl.ds(k * SUP, SUP)]],
                gat[b].at[pl.ds(k * SUP, SUP)], gsem[b])

    def wait_gathers(b):
        for k in range(8):
            pltpu.make_async_copy(dvf_ref.at[pl.ds(0, SUP)],
                                  gat[b].at[pl.ds(k * SUP, SUP)], gsem[b]).wait()

    def combine(s, b):
        @pl.loop(0, LINES)
        def _combine(li):
            for v in range(VPL):
                o = li * D + v * L
                cv = [gat[b][pl.ds(k * SUP + o, L)] for k in range(8)]
                yw = ywv[b][pl.ds(o, L)]
                xw = xwv[b][pl.ds(o, L)]
                zw = zwv[b][pl.ds(o, L)]
                c00 = cv[0] + zw * (cv[1] - cv[0])
                c01 = cv[2] + zw * (cv[3] - cv[2])
                c10 = cv[4] + zw * (cv[5] - cv[4])
                c11 = cv[6] + zw * (cv[7] - cv[6])
                c0 = c00 + xw * (c01 - c00)
                c1 = c10 + xw * (c11 - c10)
                out_v[b][pl.ds(o, L)] = c0 + yw * (c1 - c0)

        pltpu.async_copy(out_v[b], out_ref.at[pl.ds(base + s * SUP, SUP)],
                         osem[b])

    def wait_out(b):
        pltpu.make_async_copy(dvf_ref.at[pl.ds(0, SUP)], out_v[b],
                              osem[b]).wait()

    fire_dvf(0, 0)

    @pl.loop(0, NSUP // 2 + 1)
    def _t(t):
        for b in range(2):
            s = t * 2 + b

            @pl.when(s < NSUP)
            def _launch():
                wait_dvf(b)
                compute_idx(s, b)
                fire_gathers(b)

            @pl.when(s + 1 < NSUP)
            def _prefetch():
                fire_dvf(s + 1, 1 - b)

            @pl.when(s >= 3)
            def _drain_out():
                wait_out(1 - b)

            @pl.when(jnp.logical_and(s >= 1, s - 1 < NSUP))
            def _finish():
                wait_gathers(1 - b)
                combine(s - 1, 1 - b)

    wait_out((NSUP - 1) & 1)


def _mk_scratch():
    return [
        [pltpu.VMEM((SUP,), jnp.float32)] * 2,
        [pltpu.VMEM((SUP,), jnp.float32)] * 2,
        [pltpu.VMEM((SUP,), jnp.float32)] * 2,
        [pltpu.VMEM((8 * SUP,), jnp.int32)] * 2,
        [pltpu.VMEM((SUP,), jnp.float32)] * 2,
        [pltpu.VMEM((SUP,), jnp.float32)] * 2,
        [pltpu.VMEM((SUP,), jnp.float32)] * 2,
        [pltpu.VMEM((8 * SUP,), jnp.float32)] * 2,
        [pltpu.VMEM((SUP,), jnp.float32)] * 2,
        pltpu.SemaphoreType.DMA,
        [pltpu.SemaphoreType.DMA] * 2,
        [pltpu.SemaphoreType.DMA] * 2,
    ]


_warp = pl.kernel(
    _body,
    out_type=jax.ShapeDtypeStruct((VOX,), jnp.float32),
    mesh=plsc.VectorSubcoreMesh(core_axis_name="c", subcore_axis_name="s"),
    scratch_types=_mk_scratch(),
)


def kernel(image, dvf):
    flat = image.reshape(VOX)
    dvf_flat = dvf.reshape(3 * VOX)
    out = _warp(flat, dvf_flat)
    return out.reshape(1, 1, 1, H, W, D)
